# baseline (speedup 1.0000x reference)
import jax
import jax.numpy as jnp
from jax.experimental import pallas as pl
from jax.experimental.pallas import tpu as pltpu

S, D, H, E, TOPK = 2048, 1024, 16, 8, 2
DFF = 4 * D
HD = D // H

BN_QKV = 512
BQ = 256
BK = 256
BF = 512


def _gating_kernel(xl_ref, wg_ref, bg_ref, sel_ref, w_ref):
    logits = (
        jnp.dot(xl_ref[...], wg_ref[...], preferred_element_type=jnp.float32)
        + bg_ref[...]
    )
    col = jax.lax.broadcasted_iota(jnp.int32, (1, E), 1)

    def argmax_first(v):
        m = jnp.max(v, axis=1, keepdims=True)
        i = jnp.min(jnp.where(v == m, col, E), axis=1, keepdims=True)
        return m, i

    _, i0 = argmax_first(logits)
    _, i1 = argmax_first(jnp.where(col == i0, -1e30, logits))
    mask = jnp.logical_or(col == i0, col == i1).astype(jnp.float32)
    ml = logits * mask
    mx = jnp.max(ml, axis=1, keepdims=True)
    ex = jnp.exp(ml - mx)
    w = ex / jnp.sum(ex, axis=1, keepdims=True)
    w0, j0 = argmax_first(w)
    w1, j1 = argmax_first(jnp.where(col == j0, -1.0, w))
    sel_ref[...] = jnp.where(col == 0, j0, jnp.where(col == 1, j1, 0)).astype(
        jnp.int32
    )
    w_ref[...] = jnp.where(col == 0, w0, jnp.where(col == 1, w1, 0.0))


def _qkv_kernel(sel_ref, w_ref, x_ref, g_ref, b_ref, wq_ref, bq_ref, out_ref, h_scr):
    del sel_ref, w_ref
    n = pl.program_id(1)

    @pl.when(n == 0)
    def _():
        xv = x_ref[...]
        mu = jnp.mean(xv, axis=1, keepdims=True)
        xc = xv - mu
        var = jnp.mean(xc * xc, axis=1, keepdims=True)
        h = xc / jnp.sqrt(var + 1e-5)
        h_scr[...] = (h * g_ref[0] + b_ref[0]).astype(jnp.bfloat16)

    out_ref[0] = (
        jnp.dot(h_scr[...], wq_ref[0].astype(jnp.bfloat16), preferred_element_type=jnp.float32)
        + bq_ref[0]
    ).astype(jnp.bfloat16)


def _attn_kernel(q_ref, k_ref, v_ref, o_ref):
    i = pl.program_id(1)
    br = jax.lax.broadcasted_iota(jnp.int32, (BQ, S), 0) + i * BQ
    bc = jax.lax.broadcasted_iota(jnp.int32, (BQ, S), 1)
    neg = jnp.float32(-1e9)
    for h in range(H):
        lo, hi = h * HD, (h + 1) * HD
        q = q_ref[0, :, lo:hi]
        k = k_ref[0, :, lo:hi]
        s = jax.lax.dot_general(
            q, k, (((1,), (1,)), ((), ())), preferred_element_type=jnp.float32
        ) * (1.0 / 8.0)
        s = jnp.where(bc <= br, s, neg)
        m = jnp.max(s, axis=1, keepdims=True)
        p = jnp.exp(s - m)
        p = (p / jnp.sum(p, axis=1, keepdims=True)).astype(jnp.bfloat16)
        o_ref[0, :, lo:hi] = jnp.dot(
            p, v_ref[0, :, lo:hi], preferred_element_type=jnp.float32
        ).astype(jnp.bfloat16)


def _proj_kernel(sel_ref, w_ref, x_ref, y_ref, wo_ref, bo_ref, x1_ref):
    del sel_ref, w_ref
    x1_ref[0] = (
        x_ref[...]
        + jnp.dot(
            y_ref[0],
            wo_ref[0].astype(jnp.bfloat16),
            preferred_element_type=jnp.float32,
        )
        + bo_ref[0]
    )


def _ffn_kernel(
    sel_ref, w_ref, x1_ref, g_ref, b_ref, wfc_ref, bfc_ref, wp_ref, bp_ref,
    out_ref, h2_scr,
):
    del sel_ref
    j = pl.program_id(0)
    n = pl.program_id(1)
    wj = w_ref[j]

    @pl.when(jnp.logical_and(j == 0, n == 0))
    def _():
        out_ref[...] = jnp.zeros_like(out_ref)

    @pl.when(n == 0)
    def _():
        xv = x1_ref[0]
        mu = jnp.mean(xv, axis=1, keepdims=True)
        xc = xv - mu
        var = jnp.mean(xc * xc, axis=1, keepdims=True)
        h2 = xc / jnp.sqrt(var + 1e-5)
        h2_scr[...] = (h2 * g_ref[0] + b_ref[0]).astype(jnp.bfloat16)
        out_ref[...] += wj * (xv + bp_ref[0])

    m = (
        jnp.dot(h2_scr[...], wfc_ref[0].astype(jnp.bfloat16), preferred_element_type=jnp.float32)
        + bfc_ref[0]
    )
    m = 0.5 * m * (1.0 + jax.lax.erf(m * 0.7071067811865476))
    out_ref[...] += wj * jnp.dot(
        m.astype(jnp.bfloat16), wp_ref[0].astype(jnp.bfloat16), preferred_element_type=jnp.float32
    )


def kernel(x, ln1_g, ln1_b, Wqkv, bqkv, Wo, bo, ln2_g, ln2_b, Wfc, bfc, Wproj, bproj, Wg, bg):
    x2 = x.reshape(S, D)
    xl = x2[S - 1:]

    g1 = ln1_g.reshape(E, 1, D)
    b1 = ln1_b.reshape(E, 1, D)
    bqkv3 = bqkv.reshape(E, 1, 3 * D)
    bo3 = bo.reshape(E, 1, D)
    g2 = ln2_g.reshape(E, 1, D)
    b2 = ln2_b.reshape(E, 1, D)
    bfc3 = bfc.reshape(E, 1, DFF)
    bp3 = bproj.reshape(E, 1, D)

    sel8, w8 = pl.pallas_call(
        _gating_kernel,
        grid=(1,),
        in_specs=[
            pl.BlockSpec((1, D), lambda i: (0, 0)),
            pl.BlockSpec((D, E), lambda i: (0, 0)),
            pl.BlockSpec((1, E), lambda i: (0, 0)),
        ],
        out_specs=[
            pl.BlockSpec((1, E), lambda i: (0, 0)),
            pl.BlockSpec((1, E), lambda i: (0, 0)),
        ],
        out_shape=[
            jax.ShapeDtypeStruct((1, E), jnp.int32),
            jax.ShapeDtypeStruct((1, E), jnp.float32),
        ],
    )(xl, Wg, bg.reshape(1, E))

    sel = sel8[0, :TOPK]
    wts = w8[0, :TOPK]

    nn = (3 * D) // BN_QKV
    qkv = pl.pallas_call(
        _qkv_kernel,
        grid_spec=pltpu.PrefetchScalarGridSpec(
            num_scalar_prefetch=2,
            grid=(TOPK, nn),
            in_specs=[
                pl.BlockSpec((S, D), lambda j, n, sel, w: (0, 0)),
                pl.BlockSpec((1, 1, D), lambda j, n, sel, w: (sel[j], 0, 0)),
                pl.BlockSpec((1, 1, D), lambda j, n, sel, w: (sel[j], 0, 0)),
                pl.BlockSpec((1, D, BN_QKV), lambda j, n, sel, w: (sel[j], 0, n)),
                pl.BlockSpec((1, 1, BN_QKV), lambda j, n, sel, w: (sel[j], 0, n)),
            ],
            out_specs=pl.BlockSpec((1, S, BN_QKV), lambda j, n, sel, w: (j, 0, n)),
            scratch_shapes=[pltpu.VMEM((S, D), jnp.bfloat16)],
        ),
        out_shape=jax.ShapeDtypeStruct((TOPK, S, 3 * D), jnp.bfloat16),
    )(sel, wts, x2, g1, b1, Wqkv, bqkv3)

    nq = S // BQ
    y = pl.pallas_call(
        _attn_kernel,
        grid=(TOPK, nq),
        in_specs=[
            pl.BlockSpec((1, BQ, D), lambda j, i: (j, i, 0)),
            pl.BlockSpec((1, S, D), lambda j, i: (j, 0, 1)),
            pl.BlockSpec((1, S, D), lambda j, i: (j, 0, 2)),
        ],
        out_specs=pl.BlockSpec((1, BQ, D), lambda j, i: (j, i, 0)),
        out_shape=jax.ShapeDtypeStruct((TOPK, S, D), jnp.bfloat16),
    )(qkv, qkv, qkv)

    x1 = pl.pallas_call(
        _proj_kernel,
        grid_spec=pltpu.PrefetchScalarGridSpec(
            num_scalar_prefetch=2,
            grid=(TOPK,),
            in_specs=[
                pl.BlockSpec((S, D), lambda j, sel, w: (0, 0)),
                pl.BlockSpec((1, S, D), lambda j, sel, w: (j, 0, 0)),
                pl.BlockSpec((1, D, D), lambda j, sel, w: (sel[j], 0, 0)),
                pl.BlockSpec((1, 1, D), lambda j, sel, w: (sel[j], 0, 0)),
            ],
            out_specs=pl.BlockSpec((1, S, D), lambda j, sel, w: (j, 0, 0)),
        ),
        out_shape=jax.ShapeDtypeStruct((TOPK, S, D), jnp.float32),
    )(sel, wts, x2, y, Wo, bo3)

    nf = DFF // BF
    out = pl.pallas_call(
        _ffn_kernel,
        grid_spec=pltpu.PrefetchScalarGridSpec(
            num_scalar_prefetch=2,
            grid=(TOPK, nf),
            in_specs=[
                pl.BlockSpec((1, S, D), lambda j, n, sel, w: (j, 0, 0)),
                pl.BlockSpec((1, 1, D), lambda j, n, sel, w: (sel[j], 0, 0)),
                pl.BlockSpec((1, 1, D), lambda j, n, sel, w: (sel[j], 0, 0)),
                pl.BlockSpec((1, D, BF), lambda j, n, sel, w: (sel[j], 0, n)),
                pl.BlockSpec((1, 1, BF), lambda j, n, sel, w: (sel[j], 0, n)),
                pl.BlockSpec((1, BF, D), lambda j, n, sel, w: (sel[j], n, 0)),
                pl.BlockSpec((1, 1, D), lambda j, n, sel, w: (sel[j], 0, 0)),
            ],
            out_specs=pl.BlockSpec((S, D), lambda j, n, sel, w: (0, 0)),
            scratch_shapes=[pltpu.VMEM((S, D), jnp.bfloat16)],
        ),
        out_shape=jax.ShapeDtypeStruct((S, D), jnp.float32),
    )(sel, wts, x1, g2, b2, Wfc, bfc3, Wproj, bp3)

    return out.reshape(1, S, D)


# speedup vs baseline: 1.2094x; 1.2094x over previous
import jax
import jax.numpy as jnp
from jax.experimental import pallas as pl
from jax.experimental.pallas import tpu as pltpu

S, D, H, E, TOPK = 2048, 1024, 16, 8, 2
DFF = 4 * D
HD = D // H

BN_QKV = 512
BQ = 256
BK = 256
BF = 512


def _gating_kernel(xl_ref, wg_ref, bg_ref, sel_ref, w_ref):
    logits = (
        jnp.dot(xl_ref[...], wg_ref[...], preferred_element_type=jnp.float32)
        + bg_ref[...]
    )
    col = jax.lax.broadcasted_iota(jnp.int32, (1, E), 1)

    def argmax_first(v):
        m = jnp.max(v, axis=1, keepdims=True)
        i = jnp.min(jnp.where(v == m, col, E), axis=1, keepdims=True)
        return m, i

    _, i0 = argmax_first(logits)
    _, i1 = argmax_first(jnp.where(col == i0, -1e30, logits))
    mask = jnp.logical_or(col == i0, col == i1).astype(jnp.float32)
    ml = logits * mask
    mx = jnp.max(ml, axis=1, keepdims=True)
    ex = jnp.exp(ml - mx)
    w = ex / jnp.sum(ex, axis=1, keepdims=True)
    w0, j0 = argmax_first(w)
    w1, j1 = argmax_first(jnp.where(col == j0, -1.0, w))
    sel_ref[...] = jnp.where(col == 0, j0, jnp.where(col == 1, j1, 0)).astype(
        jnp.int32
    )
    w_ref[...] = jnp.where(col == 0, w0, jnp.where(col == 1, w1, 0.0))


def _qkv_kernel(sel_ref, w_ref, x_ref, g_ref, b_ref, wq_ref, bq_ref, out_ref, h_scr):
    del sel_ref, w_ref
    n = pl.program_id(1)

    @pl.when(n == 0)
    def _():
        xv = x_ref[...]
        mu = jnp.mean(xv, axis=1, keepdims=True)
        xc = xv - mu
        var = jnp.mean(xc * xc, axis=1, keepdims=True)
        h = xc / jnp.sqrt(var + 1e-5)
        h_scr[...] = (h * g_ref[0] + b_ref[0]).astype(jnp.bfloat16)

    out_ref[0] = (
        jnp.dot(h_scr[...], wq_ref[0].astype(jnp.bfloat16), preferred_element_type=jnp.float32)
        + bq_ref[0]
    )


def _attn_kernel(q_ref, k_ref, v_ref, o_ref):
    i = pl.program_id(1)
    br = jax.lax.broadcasted_iota(jnp.int32, (BQ, S), 0) + i * BQ
    bc = jax.lax.broadcasted_iota(jnp.int32, (BQ, S), 1)
    neg = jnp.float32(-1e9)
    for h in range(H):
        lo, hi = h * HD, (h + 1) * HD
        q = q_ref[0, :, lo:hi]
        k = k_ref[0, :, lo:hi]
        s = jax.lax.dot_general(
            q, k, (((1,), (1,)), ((), ())), preferred_element_type=jnp.float32
        ) * (1.0 / 8.0)
        s = jnp.where(bc <= br, s, neg)
        m = jnp.max(s, axis=1, keepdims=True)
        p = jnp.exp(s - m)
        l = jnp.sum(p, axis=1, keepdims=True)
        o = jnp.dot(p, v_ref[0, :, lo:hi], preferred_element_type=jnp.float32)
        o_ref[0, :, lo:hi] = o * (1.0 / l)


def _proj_kernel(sel_ref, w_ref, x_ref, y_ref, wo_ref, bo_ref, x1_ref):
    del sel_ref, w_ref
    x1_ref[0] = (
        x_ref[...]
        + jnp.dot(
            y_ref[0].astype(jnp.bfloat16),
            wo_ref[0].astype(jnp.bfloat16),
            preferred_element_type=jnp.float32,
        )
        + bo_ref[0]
    )


def _ffn_kernel(
    sel_ref, w_ref, x1_ref, g_ref, b_ref, wfc_ref, bfc_ref, wp_ref, bp_ref,
    out_ref, h2_scr,
):
    del sel_ref
    j = pl.program_id(0)
    n = pl.program_id(1)
    wj = w_ref[j]

    @pl.when(jnp.logical_and(j == 0, n == 0))
    def _():
        out_ref[...] = jnp.zeros_like(out_ref)

    @pl.when(n == 0)
    def _():
        xv = x1_ref[0]
        mu = jnp.mean(xv, axis=1, keepdims=True)
        xc = xv - mu
        var = jnp.mean(xc * xc, axis=1, keepdims=True)
        h2 = xc / jnp.sqrt(var + 1e-5)
        h2_scr[...] = (h2 * g_ref[0] + b_ref[0]).astype(jnp.bfloat16)
        out_ref[...] += wj * (xv + bp_ref[0])

    m = (
        jnp.dot(h2_scr[...], wfc_ref[0].astype(jnp.bfloat16), preferred_element_type=jnp.float32)
        + bfc_ref[0]
    )
    m = 0.5 * m * (1.0 + jax.lax.erf(m * 0.7071067811865476))
    out_ref[...] += wj * jnp.dot(
        m.astype(jnp.bfloat16), wp_ref[0].astype(jnp.bfloat16), preferred_element_type=jnp.float32
    )


def kernel(x, ln1_g, ln1_b, Wqkv, bqkv, Wo, bo, ln2_g, ln2_b, Wfc, bfc, Wproj, bproj, Wg, bg):
    x2 = x.reshape(S, D)
    xl = x2[S - 1:]

    g1 = ln1_g.reshape(E, 1, D)
    b1 = ln1_b.reshape(E, 1, D)
    bqkv3 = bqkv.reshape(E, 1, 3 * D)
    bo3 = bo.reshape(E, 1, D)
    g2 = ln2_g.reshape(E, 1, D)
    b2 = ln2_b.reshape(E, 1, D)
    bfc3 = bfc.reshape(E, 1, DFF)
    bp3 = bproj.reshape(E, 1, D)

    sel8, w8 = pl.pallas_call(
        _gating_kernel,
        grid=(1,),
        in_specs=[
            pl.BlockSpec((1, D), lambda i: (0, 0)),
            pl.BlockSpec((D, E), lambda i: (0, 0)),
            pl.BlockSpec((1, E), lambda i: (0, 0)),
        ],
        out_specs=[
            pl.BlockSpec((1, E), lambda i: (0, 0)),
            pl.BlockSpec((1, E), lambda i: (0, 0)),
        ],
        out_shape=[
            jax.ShapeDtypeStruct((1, E), jnp.int32),
            jax.ShapeDtypeStruct((1, E), jnp.float32),
        ],
    )(xl, Wg, bg.reshape(1, E))

    sel = sel8[0, :TOPK]
    wts = w8[0, :TOPK]

    nn = (3 * D) // BN_QKV
    qkv = pl.pallas_call(
        _qkv_kernel,
        grid_spec=pltpu.PrefetchScalarGridSpec(
            num_scalar_prefetch=2,
            grid=(TOPK, nn),
            in_specs=[
                pl.BlockSpec((S, D), lambda j, n, sel, w: (0, 0)),
                pl.BlockSpec((1, 1, D), lambda j, n, sel, w: (sel[j], 0, 0)),
                pl.BlockSpec((1, 1, D), lambda j, n, sel, w: (sel[j], 0, 0)),
                pl.BlockSpec((1, D, BN_QKV), lambda j, n, sel, w: (sel[j], 0, n)),
                pl.BlockSpec((1, 1, BN_QKV), lambda j, n, sel, w: (sel[j], 0, n)),
            ],
            out_specs=pl.BlockSpec((1, S, BN_QKV), lambda j, n, sel, w: (j, 0, n)),
            scratch_shapes=[pltpu.VMEM((S, D), jnp.bfloat16)],
        ),
        out_shape=jax.ShapeDtypeStruct((TOPK, S, 3 * D), jnp.float32),
    )(sel, wts, x2, g1, b1, Wqkv, bqkv3)

    nq = S // BQ
    y = pl.pallas_call(
        _attn_kernel,
        grid=(TOPK, nq),
        in_specs=[
            pl.BlockSpec((1, BQ, D), lambda j, i: (j, i, 0)),
            pl.BlockSpec((1, S, D), lambda j, i: (j, 0, 1)),
            pl.BlockSpec((1, S, D), lambda j, i: (j, 0, 2)),
        ],
        out_specs=pl.BlockSpec((1, BQ, D), lambda j, i: (j, i, 0)),
        out_shape=jax.ShapeDtypeStruct((TOPK, S, D), jnp.float32),
    )(qkv, qkv, qkv)

    x1 = pl.pallas_call(
        _proj_kernel,
        grid_spec=pltpu.PrefetchScalarGridSpec(
            num_scalar_prefetch=2,
            grid=(TOPK,),
            in_specs=[
                pl.BlockSpec((S, D), lambda j, sel, w: (0, 0)),
                pl.BlockSpec((1, S, D), lambda j, sel, w: (j, 0, 0)),
                pl.BlockSpec((1, D, D), lambda j, sel, w: (sel[j], 0, 0)),
                pl.BlockSpec((1, 1, D), lambda j, sel, w: (sel[j], 0, 0)),
            ],
            out_specs=pl.BlockSpec((1, S, D), lambda j, sel, w: (j, 0, 0)),
        ),
        out_shape=jax.ShapeDtypeStruct((TOPK, S, D), jnp.float32),
    )(sel, wts, x2, y, Wo, bo3)

    nf = DFF // BF
    out = pl.pallas_call(
        _ffn_kernel,
        grid_spec=pltpu.PrefetchScalarGridSpec(
            num_scalar_prefetch=2,
            grid=(TOPK, nf),
            in_specs=[
                pl.BlockSpec((1, S, D), lambda j, n, sel, w: (j, 0, 0)),
                pl.BlockSpec((1, 1, D), lambda j, n, sel, w: (sel[j], 0, 0)),
                pl.BlockSpec((1, 1, D), lambda j, n, sel, w: (sel[j], 0, 0)),
                pl.BlockSpec((1, D, BF), lambda j, n, sel, w: (sel[j], 0, n)),
                pl.BlockSpec((1, 1, BF), lambda j, n, sel, w: (sel[j], 0, n)),
                pl.BlockSpec((1, BF, D), lambda j, n, sel, w: (sel[j], n, 0)),
                pl.BlockSpec((1, 1, D), lambda j, n, sel, w: (sel[j], 0, 0)),
            ],
            out_specs=pl.BlockSpec((S, D), lambda j, n, sel, w: (0, 0)),
            scratch_shapes=[pltpu.VMEM((S, D), jnp.bfloat16)],
        ),
        out_shape=jax.ShapeDtypeStruct((S, D), jnp.float32),
    )(sel, wts, x1, g2, b2, Wfc, bfc3, Wproj, bp3)

    return out.reshape(1, S, D)


# speedup vs baseline: 1.3015x; 1.0762x over previous
import jax
import jax.numpy as jnp
from jax.experimental import pallas as pl
from jax.experimental.pallas import tpu as pltpu

S, D, H, E, TOPK = 2048, 1024, 16, 8, 2
DFF = 4 * D
HD = D // H

BN_QKV = 512
BQ = 256
BK = 256
BF = 512


def _gating_kernel(xl_ref, wg_ref, bg_ref, sel_ref, w_ref):
    logits = (
        jnp.dot(xl_ref[...], wg_ref[...], preferred_element_type=jnp.float32)
        + bg_ref[...]
    )
    col = jax.lax.broadcasted_iota(jnp.int32, (1, E), 1)

    def argmax_first(v):
        m = jnp.max(v, axis=1, keepdims=True)
        i = jnp.min(jnp.where(v == m, col, E), axis=1, keepdims=True)
        return m, i

    _, i0 = argmax_first(logits)
    _, i1 = argmax_first(jnp.where(col == i0, -1e30, logits))
    mask = jnp.logical_or(col == i0, col == i1).astype(jnp.float32)
    ml = logits * mask
    mx = jnp.max(ml, axis=1, keepdims=True)
    ex = jnp.exp(ml - mx)
    w = ex / jnp.sum(ex, axis=1, keepdims=True)
    w0, j0 = argmax_first(w)
    w1, j1 = argmax_first(jnp.where(col == j0, -1.0, w))
    sel_ref[...] = jnp.where(col == 0, j0, jnp.where(col == 1, j1, 0)).astype(
        jnp.int32
    )
    w_ref[...] = jnp.where(col == 0, w0, jnp.where(col == 1, w1, 0.0))


def _qkv_kernel(sel_ref, w_ref, x_ref, g_ref, b_ref, wq_ref, bq_ref, out_ref, h_scr):
    del sel_ref, w_ref
    n = pl.program_id(1)

    @pl.when(n == 0)
    def _():
        xv = x_ref[...]
        mu = jnp.mean(xv, axis=1, keepdims=True)
        xc = xv - mu
        var = jnp.mean(xc * xc, axis=1, keepdims=True)
        h = xc / jnp.sqrt(var + 1e-5)
        h_scr[...] = (h * g_ref[0] + b_ref[0]).astype(jnp.bfloat16)

    out_ref[0] = (
        jnp.dot(h_scr[...], wq_ref[0].astype(jnp.bfloat16), preferred_element_type=jnp.float32)
        + bq_ref[0]
    )


def _make_attn_kernel(tile0, ntile, W):
    def _attn_kernel(q_ref, k_ref, v_ref, o_ref):
        i = pl.program_id(1)
        br = jax.lax.broadcasted_iota(jnp.int32, (BQ, W), 0) + (i + tile0) * BQ
        bc = jax.lax.broadcasted_iota(jnp.int32, (BQ, W), 1)
        neg = jnp.float32(-1e9)
        for h in range(H):
            lo, hi = h * HD, (h + 1) * HD
            q = q_ref[0, :, lo:hi]
            k = k_ref[0, :, lo:hi]
            s = jax.lax.dot_general(
                q, k, (((1,), (1,)), ((), ())),
                preferred_element_type=jnp.float32,
            ) * (1.0 / 8.0)
            s = jnp.where(bc <= br, s, neg)
            m = jnp.max(s, axis=1, keepdims=True)
            p = jnp.exp(s - m)
            l = jnp.sum(p, axis=1, keepdims=True)
            o = jnp.dot(
                p, v_ref[0, :, lo:hi], preferred_element_type=jnp.float32
            )
            o_ref[0, :, lo:hi] = o * (1.0 / l)

    return _attn_kernel


def _proj_kernel(sel_ref, w_ref, x_ref, y_ref, wo_ref, bo_ref, x1_ref):
    del sel_ref, w_ref
    x1_ref[0] = (
        x_ref[...]
        + jnp.dot(
            y_ref[0].astype(jnp.bfloat16),
            wo_ref[0].astype(jnp.bfloat16),
            preferred_element_type=jnp.float32,
        )
        + bo_ref[0]
    )


def _ffn_kernel(
    sel_ref, w_ref, x1_ref, g_ref, b_ref, wfc_ref, bfc_ref, wp_ref, bp_ref,
    out_ref, h2_scr,
):
    del sel_ref
    j = pl.program_id(0)
    n = pl.program_id(1)
    wj = w_ref[j]

    @pl.when(jnp.logical_and(j == 0, n == 0))
    def _():
        out_ref[...] = jnp.zeros_like(out_ref)

    @pl.when(n == 0)
    def _():
        xv = x1_ref[0]
        mu = jnp.mean(xv, axis=1, keepdims=True)
        xc = xv - mu
        var = jnp.mean(xc * xc, axis=1, keepdims=True)
        h2 = xc / jnp.sqrt(var + 1e-5)
        h2_scr[...] = (h2 * g_ref[0] + b_ref[0]).astype(jnp.bfloat16)
        out_ref[...] += wj * (xv + bp_ref[0])

    m = (
        jnp.dot(h2_scr[...], wfc_ref[0].astype(jnp.bfloat16), preferred_element_type=jnp.float32)
        + bfc_ref[0]
    )
    m = 0.5 * m * (1.0 + jax.lax.erf(m * 0.7071067811865476))
    out_ref[...] += wj * jnp.dot(
        m.astype(jnp.bfloat16), wp_ref[0].astype(jnp.bfloat16), preferred_element_type=jnp.float32
    )


def kernel(x, ln1_g, ln1_b, Wqkv, bqkv, Wo, bo, ln2_g, ln2_b, Wfc, bfc, Wproj, bproj, Wg, bg):
    x2 = x.reshape(S, D)
    xl = x2[S - 1:]

    g1 = ln1_g.reshape(E, 1, D)
    b1 = ln1_b.reshape(E, 1, D)
    bqkv3 = bqkv.reshape(E, 1, 3 * D)
    bo3 = bo.reshape(E, 1, D)
    g2 = ln2_g.reshape(E, 1, D)
    b2 = ln2_b.reshape(E, 1, D)
    bfc3 = bfc.reshape(E, 1, DFF)
    bp3 = bproj.reshape(E, 1, D)

    sel8, w8 = pl.pallas_call(
        _gating_kernel,
        grid=(1,),
        in_specs=[
            pl.BlockSpec((1, D), lambda i: (0, 0)),
            pl.BlockSpec((D, E), lambda i: (0, 0)),
            pl.BlockSpec((1, E), lambda i: (0, 0)),
        ],
        out_specs=[
            pl.BlockSpec((1, E), lambda i: (0, 0)),
            pl.BlockSpec((1, E), lambda i: (0, 0)),
        ],
        out_shape=[
            jax.ShapeDtypeStruct((1, E), jnp.int32),
            jax.ShapeDtypeStruct((1, E), jnp.float32),
        ],
    )(xl, Wg, bg.reshape(1, E))

    sel = sel8[0, :TOPK]
    wts = w8[0, :TOPK]

    nn = (3 * D) // BN_QKV
    qkv = pl.pallas_call(
        _qkv_kernel,
        grid_spec=pltpu.PrefetchScalarGridSpec(
            num_scalar_prefetch=2,
            grid=(TOPK, nn),
            in_specs=[
                pl.BlockSpec((S, D), lambda j, n, sel, w: (0, 0)),
                pl.BlockSpec((1, 1, D), lambda j, n, sel, w: (sel[j], 0, 0)),
                pl.BlockSpec((1, 1, D), lambda j, n, sel, w: (sel[j], 0, 0)),
                pl.BlockSpec((1, D, BN_QKV), lambda j, n, sel, w: (sel[j], 0, n)),
                pl.BlockSpec((1, 1, BN_QKV), lambda j, n, sel, w: (sel[j], 0, n)),
            ],
            out_specs=pl.BlockSpec((1, S, BN_QKV), lambda j, n, sel, w: (j, 0, n)),
            scratch_shapes=[pltpu.VMEM((S, D), jnp.bfloat16)],
        ),
        out_shape=jax.ShapeDtypeStruct((TOPK, S, 3 * D), jnp.float32),
    )(sel, wts, x2, g1, b1, Wqkv, bqkv3)

    y_parts = []
    for tile0, ntile, W in ((0, 2, 2 * BQ), (2, 2, 4 * BQ), (4, 4, S)):
        y_parts.append(
            pl.pallas_call(
                _make_attn_kernel(tile0, ntile, W),
                grid=(TOPK, ntile),
                in_specs=[
                    pl.BlockSpec(
                        (1, BQ, D), lambda j, i, t0=tile0: (j, i + t0, 0)
                    ),
                    pl.BlockSpec((1, W, D), lambda j, i: (j, 0, 1)),
                    pl.BlockSpec((1, W, D), lambda j, i: (j, 0, 2)),
                ],
                out_specs=pl.BlockSpec((1, BQ, D), lambda j, i: (j, i, 0)),
                out_shape=jax.ShapeDtypeStruct(
                    (TOPK, ntile * BQ, D), jnp.float32
                ),
            )(qkv, qkv, qkv)
        )
    y = jnp.concatenate(y_parts, axis=1)

    x1 = pl.pallas_call(
        _proj_kernel,
        grid_spec=pltpu.PrefetchScalarGridSpec(
            num_scalar_prefetch=2,
            grid=(TOPK,),
            in_specs=[
                pl.BlockSpec((S, D), lambda j, sel, w: (0, 0)),
                pl.BlockSpec((1, S, D), lambda j, sel, w: (j, 0, 0)),
                pl.BlockSpec((1, D, D), lambda j, sel, w: (sel[j], 0, 0)),
                pl.BlockSpec((1, 1, D), lambda j, sel, w: (sel[j], 0, 0)),
            ],
            out_specs=pl.BlockSpec((1, S, D), lambda j, sel, w: (j, 0, 0)),
        ),
        out_shape=jax.ShapeDtypeStruct((TOPK, S, D), jnp.float32),
    )(sel, wts, x2, y, Wo, bo3)

    nf = DFF // BF
    out = pl.pallas_call(
        _ffn_kernel,
        grid_spec=pltpu.PrefetchScalarGridSpec(
            num_scalar_prefetch=2,
            grid=(TOPK, nf),
            in_specs=[
                pl.BlockSpec((1, S, D), lambda j, n, sel, w: (j, 0, 0)),
                pl.BlockSpec((1, 1, D), lambda j, n, sel, w: (sel[j], 0, 0)),
                pl.BlockSpec((1, 1, D), lambda j, n, sel, w: (sel[j], 0, 0)),
                pl.BlockSpec((1, D, BF), lambda j, n, sel, w: (sel[j], 0, n)),
                pl.BlockSpec((1, 1, BF), lambda j, n, sel, w: (sel[j], 0, n)),
                pl.BlockSpec((1, BF, D), lambda j, n, sel, w: (sel[j], n, 0)),
                pl.BlockSpec((1, 1, D), lambda j, n, sel, w: (sel[j], 0, 0)),
            ],
            out_specs=pl.BlockSpec((S, D), lambda j, n, sel, w: (0, 0)),
            scratch_shapes=[pltpu.VMEM((S, D), jnp.bfloat16)],
        ),
        out_shape=jax.ShapeDtypeStruct((S, D), jnp.float32),
    )(sel, wts, x1, g2, b2, Wfc, bfc3, Wproj, bp3)

    return out.reshape(1, S, D)
